# baseline (speedup 1.0000x reference)
import functools

import jax
import jax.numpy as jnp
from jax import lax
from jax.experimental import pallas as pl
from jax.experimental.pallas import tpu as pltpu
from jax.experimental.pallas import tpu_sc as plsc

_INFO = plsc.get_sparse_core_info()
_NC = _INFO.num_cores
_NS = _INFO.num_subcores
_NW = _NC * _NS
_C = 128
_L = 16


def _mlp_t_body(gft_ref, w1t_ref, b1_ref, w2t_ref, b2_ref, gt_ref):
    h = jnp.dot(w1t_ref[...], gft_ref[...], preferred_element_type=jnp.float32)
    h = jnp.maximum(h + b1_ref[...], 0.0)
    g = jnp.dot(w2t_ref[...], h, preferred_element_type=jnp.float32)
    gt_ref[...] = g + b2_ref[...]


def _run_mlp_t(graph_feat, W1, b1, W2, b2):
    G = graph_feat.shape[0]
    O = W2.shape[1]
    return pl.pallas_call(
        _mlp_t_body,
        out_shape=jax.ShapeDtypeStruct((O, G), jnp.float32),
    )(graph_feat.T, W1.T, b1.reshape(-1, 1), W2.T, b2.reshape(-1, 1))


def _make_gather_kernel(N_pad, E, G, D):
    assert N_pad % _C == 0 and E % _C == 0
    n_ch = N_pad // _C
    e_ch = E // _C
    n_J = -(-n_ch // (2 * _NW))
    e_J = -(-e_ch // (2 * _NW))
    mesh = plsc.VectorSubcoreMesh(core_axis_name="c", subcore_axis_name="s")

    @functools.partial(
        pl.kernel,
        mesh=mesh,
        out_type=(
            jax.ShapeDtypeStruct((D, N_pad), jnp.float32),
            jax.ShapeDtypeStruct((D, E), jnp.float32),
        ),
        scratch_types=[
            pltpu.VMEM((D * G,), jnp.int32),
            pltpu.VMEM((N_pad,), jnp.int32),
            pltpu.VMEM((2, _C), jnp.int32),
            pltpu.VMEM((2, D, _C), jnp.float32),
            pltpu.SemaphoreType.DMA,
            pltpu.SemaphoreType.DMA,
            pltpu.SemaphoreType.DMA,
            pltpu.SemaphoreType.DMA,
        ],
        compiler_params=pltpu.CompilerParams(
            use_tc_tiling_on_sc=True, needs_layout_passes=False),
    )
    def gather_kernel(g_hbm, batch_hbm, ei0_hbm,
                      xt_hbm, eat_hbm,
                      g_v, b_v, eidx, out,
                      sem_w0, sem_w1, sem_i0, sem_i1):
        cid = lax.axis_index("c")
        sid = lax.axis_index("s")
        wid = sid * _NC + cid
        sem_w = (sem_w0, sem_w1)
        sem_i = (sem_i0, sem_i1)

        pltpu.sync_copy(g_hbm, g_v)
        pltpu.sync_copy(batch_hbm, b_v)

        def build_block(s, idxv_of, row0p, dst_hbm, base):
            K = 8
            groups = [(t, f0)
                      for t in range(_C // _L)
                      for f0 in range(0, D // 2, K)]
            idxvs = {}

            def issue(t, f0):
                if t not in idxvs:
                    idxvs[t] = idxv_of(t)
                idxv = idxvs[t]
                return [plsc.load_gather(
                    g_v, [idxv + (row0p + f0 + k) * G]) for k in range(K)]

            nxt = issue(*groups[0])
            for gi, (t, f0) in enumerate(groups):
                pvals = nxt
                if gi + 1 < len(groups):
                    nxt = issue(*groups[gi + 1])
                for k in range(K):
                    a, b = plsc.unpack(
                        plsc.bitcast(pvals[k], jnp.bfloat16),
                        format=plsc.PackFormat.INTERLEAVED)
                    fo = 2 * (f0 + k)
                    out.at[s][fo, pl.ds(t * _L, _L)] = a
                    out.at[s][fo + 1, pl.ds(t * _L, _L)] = b
            pltpu.async_copy(out.at[s], dst_hbm.at[:, pl.ds(base, _C)],
                             sem_w[s])

        def drain_write(s, dst_hbm, base):
            pltpu.make_async_copy(
                out.at[s], dst_hbm.at[:, pl.ds(base, _C)], sem_w[s]).wait()

        def n_body(j, carry):
            for s in range(2):
                c = wid + 32 * s + 64 * j
                base = pl.multiple_of(c * _C, _C)

                @pl.when((j > 0) & (c - 64 < n_ch))
                def _():
                    drain_write(s, xt_hbm,
                                pl.multiple_of((c - 64) * _C, _C))

                @pl.when(c < n_ch)
                def _():
                    def idxv_of(t):
                        return b_v[pl.ds(base + t * _L, _L)]
                    build_block(s, idxv_of, 0, xt_hbm, base)
            return carry

        lax.fori_loop(0, n_J, n_body, 0, unroll=False)
        for s in range(2):
            c_last = wid + 32 * s + 64 * (n_J - 1)

            @pl.when(c_last < n_ch)
            def _():
                drain_write(s, xt_hbm, pl.multiple_of(c_last * _C, _C))

        def issue_eidx(s, c):
            pltpu.async_copy(ei0_hbm.at[pl.ds(pl.multiple_of(c * _C, _C), _C)],
                             eidx.at[s], sem_i[s])

        def wait_eidx(s, c):
            pltpu.make_async_copy(
                ei0_hbm.at[pl.ds(pl.multiple_of(c * _C, _C), _C)],
                eidx.at[s], sem_i[s]).wait()

        for s in range(2):
            c0 = wid + 32 * s

            @pl.when(c0 < e_ch)
            def _():
                issue_eidx(s, c0)

        def e_body(j, carry):
            for s in range(2):
                c = wid + 32 * s + 64 * j
                base = pl.multiple_of(c * _C, _C)

                @pl.when((j > 0) & (c - 64 < e_ch))
                def _():
                    drain_write(s, eat_hbm,
                                pl.multiple_of((c - 64) * _C, _C))

                @pl.when(c < e_ch)
                def _():
                    wait_eidx(s, c)

                    def idxv_of(t):
                        srcv = eidx.at[s][pl.ds(t * _L, _L)]
                        return plsc.load_gather(b_v, [srcv])
                    build_block(s, idxv_of, D // 2, eat_hbm, base)

                @pl.when(c + 64 < e_ch)
                def _():
                    issue_eidx(s, c + 64)
            return carry

        lax.fori_loop(0, e_J, e_body, 0, unroll=False)
        for s in range(2):
            c_last = wid + 32 * s + 64 * (e_J - 1)

            @pl.when(c_last < e_ch)
            def _():
                drain_write(s, eat_hbm, pl.multiple_of(c_last * _C, _C))

    return gather_kernel


def kernel(graph_feat, batch, edge_index, W1, b1, W2, b2):
    N = batch.shape[0]
    E = edge_index.shape[1]
    G = graph_feat.shape[0]
    D = W2.shape[1] // 2

    gT = _run_mlp_t(graph_feat, W1, b1, W2, b2)
    u = jax.lax.bitcast_convert_type(
        gT.astype(jnp.bfloat16), jnp.uint16).astype(jnp.uint32)
    g_flat = jax.lax.bitcast_convert_type(
        u[0::2] | (u[1::2] << 16), jnp.int32).reshape(-1)

    N_pad = -(-N // _C) * _C
    batch_pad = jnp.concatenate(
        [batch, jnp.zeros((N_pad - N,), dtype=batch.dtype)])

    gather = _make_gather_kernel(N_pad, E, G, D)
    xt_pad, eat = gather(g_flat, batch_pad, edge_index[0])

    x = xt_pad[:, :N].T
    edge_attr = eat.T
    return (x, edge_index, edge_attr, batch)
